# baseline (speedup 1.0000x reference)
import functools

import jax
import jax.numpy as jnp
import numpy as np
from jax import lax
from jax.experimental import pallas as pl
from jax.experimental.pallas import tpu as pltpu
from jax.experimental.pallas import tpu_sc as plsc

N = 10000
E = 320000
D = 128
NB = 10
RN = 64
NG = 16
MAX_RADIUS = 5.0
INV_SQRT_NN = 1.0 / np.sqrt(32.0)

_NC = 2
_NS = 16
_NW = _NC * _NS


_LEN_CH = 2000
_EPW = E // _NW


def _edge_len2_kernel(pos9_hbm, src_hbm, dst_hbm, out_hbm, pos_v, src_v,
                      dst_v, ob0_v, ob1_v, ob2_v):
    c = lax.axis_index("c")
    s = lax.axis_index("s")
    pltpu.sync_copy(pos9_hbm, pos_v)
    nch = jnp.where(c == 0, 6, 4)
    base0 = jnp.where(c == 0, s * 6, _NS * 6 + s * 4) * _LEN_CH
    obs = (ob0_v, ob1_v, ob2_v)

    def chunk_body(ci, carry):
        base = base0 + ci * _LEN_CH
        pltpu.sync_copy(src_hbm.at[pl.ds(base, _LEN_CH)], src_v)
        pltpu.sync_copy(dst_hbm.at[pl.ds(base, _LEN_CH)], dst_v)

        def vec_body(vi, c2):
            off = vi * 16
            si = src_v[pl.ds(off, 16)]
            di = dst_v[pl.ds(off, 16)]
            for t in range(3):
                acc = jnp.zeros((16,), jnp.float32)
                for comp in range(3):
                    bias = jnp.full((16,), (t * 3 + comp) * N, jnp.int32)
                    a = plsc.load_gather(pos_v, [bias + di])
                    b = plsc.load_gather(pos_v, [bias + si])
                    dv = a - b
                    acc = acc + dv * dv
                obs[t][pl.ds(off, 16)] = acc
            return c2

        lax.fori_loop(0, _LEN_CH // 16, vec_body, 0)
        for t in range(3):
            pltpu.sync_copy(obs[t], out_hbm.at[pl.ds(t * E + base, _LEN_CH)])
        return carry

    lax.fori_loop(0, nch, chunk_body, 0)


def _edge_len2(pos9, src, dst):
    mesh = plsc.VectorSubcoreMesh(core_axis_name="c", subcore_axis_name="s")
    fn = functools.partial(
        pl.kernel,
        mesh=mesh,
        out_type=jax.ShapeDtypeStruct((3 * E,), jnp.float32),
        scratch_types=[
            pltpu.VMEM((9 * N,), jnp.float32),
            pltpu.VMEM((_LEN_CH,), jnp.int32),
            pltpu.VMEM((_LEN_CH,), jnp.int32),
            pltpu.VMEM((_LEN_CH,), jnp.float32),
            pltpu.VMEM((_LEN_CH,), jnp.float32),
            pltpu.VMEM((_LEN_CH,), jnp.float32),
        ],
        compiler_params=pltpu.CompilerParams(needs_layout_passes=False),
    )(_edge_len2_kernel)
    return fn(pos9, src, dst)


_BLK = 96
_E_PAD = 322560
_MB0 = 140
_MB1 = 70
_NPAD = 10112
_RPT = _NPAD // _NS


def _edge_agg_kernel(xw_hbm, rad_hbm, src_hbm, dst_hbm, zeros_hbm, out_hbm,
                     si0, si1, di0, di1, xr0, xr1, rd0, rd1, agg_sh,
                     sg0, sg1, ss0, ss1, sd0, sd1, sr0, sr1, sc0, sc1):
    c = lax.axis_index("c")
    s = lax.axis_index("s")
    wid = s * _NC + c
    pltpu.sync_copy(zeros_hbm.at[pl.ds(s * _RPT, _RPT)],
                    agg_sh.at[pl.ds(s * _RPT, _RPT)])
    plsc.subcore_barrier()

    si = (si0, si1)
    di = (di0, di1)
    xr = (xr0, xr1)
    rd = (rd0, rd1)
    sg = (sg0, sg1)
    ss = (ss0, ss1)
    sd = (sd0, sd1)
    sr = (sr0, sr1)
    sc = (sc0, sc1)
    mb = jnp.where(c == 0, _MB0, _MB1)
    base0 = jnp.where(c == 0, s * _MB0, _NS * _MB0 + s * _MB1) * _BLK

    def _multiply(xr_v, rd_v):
        def row_body(i, c2):
            for j in range(D // 16):
                sl = pl.ds(j * 16, 16)
                xr_v[i, sl] = xr_v[i, sl] * rd_v[i, sl]
            return c2

        lax.fori_loop(0, _BLK, row_body, 0)

    pltpu.sync_copy(src_hbm.at[pl.ds(base0, _BLK)], si0)
    pltpu.async_copy(xw_hbm.at[si0], xr0, sg0)
    pltpu.async_copy(rad_hbm.at[pl.ds(base0, _BLK)], rd0, sr0)
    pltpu.async_copy(dst_hbm.at[pl.ds(base0, _BLK)], di0, sd0)
    pltpu.async_copy(src_hbm.at[pl.ds(base0 + _BLK, _BLK)], si1, ss1)
    pltpu.async_copy(rad_hbm.at[pl.ds(base0 + _BLK, _BLK)], rd1, sr1)

    def step(kk, carry):
        for slot in (0, 1):
            nxt = slot ^ 1
            k = kk * 2 + slot
            base = base0 + k * _BLK
            pltpu.make_async_copy(xw_hbm.at[si[slot]], xr[slot],
                                  sg[slot]).wait()

            @pl.when(k + 1 < mb)
            def _():
                pltpu.make_async_copy(src_hbm.at[pl.ds(base + _BLK, _BLK)],
                                      si[nxt], ss[nxt]).wait()

                @pl.when(k >= 1)
                def _():
                    pltpu.make_async_copy(xr[nxt], agg_sh.at[di[nxt]],
                                          sc[nxt]).wait()

                pltpu.async_copy(xw_hbm.at[si[nxt]], xr[nxt], sg[nxt])
                pltpu.async_copy(dst_hbm.at[pl.ds(base + _BLK, _BLK)],
                                 di[nxt], sd[nxt])

            @pl.when(k + 2 < mb)
            def _():
                pltpu.async_copy(src_hbm.at[pl.ds(base + 2 * _BLK, _BLK)],
                                 si[slot], ss[slot])

            pltpu.make_async_copy(rad_hbm.at[pl.ds(base, _BLK)], rd[slot],
                                  sr[slot]).wait()
            _multiply(xr[slot], rd[slot])

            @pl.when(k + 2 < mb)
            def _():
                pltpu.async_copy(rad_hbm.at[pl.ds(base + 2 * _BLK, _BLK)],
                                 rd[slot], sr[slot])

            pltpu.make_async_copy(dst_hbm.at[pl.ds(base, _BLK)], di[slot],
                                  sd[slot]).wait()
            pltpu.async_copy(xr[slot], agg_sh.at[di[slot]], sc[slot],
                             add=True)

        return carry

    lax.fori_loop(0, mb // 2, step, 0)

    pltpu.make_async_copy(xr0, agg_sh.at[di0], sc0).wait()
    pltpu.make_async_copy(xr1, agg_sh.at[di1], sc1).wait()

    plsc.subcore_barrier()
    pltpu.sync_copy(agg_sh.at[pl.ds(s * _RPT, _RPT)],
                    out_hbm.at[pl.ds(c * _NPAD + s * _RPT, _RPT)])


def _edge_agg(xw, radial, src, dst, zeros_nd):
    mesh = plsc.VectorSubcoreMesh(core_axis_name="c", subcore_axis_name="s")
    fn = functools.partial(
        pl.kernel,
        mesh=mesh,
        out_type=jax.ShapeDtypeStruct((2 * _NPAD, D), jnp.float32),
        scratch_types=[
            pltpu.VMEM((_BLK,), jnp.int32),
            pltpu.VMEM((_BLK,), jnp.int32),
            pltpu.VMEM((_BLK,), jnp.int32),
            pltpu.VMEM((_BLK,), jnp.int32),
            pltpu.VMEM((_BLK, D), jnp.float32),
            pltpu.VMEM((_BLK, D), jnp.float32),
            pltpu.VMEM((_BLK, D), jnp.float32),
            pltpu.VMEM((_BLK, D), jnp.float32),
            pltpu.VMEM_SHARED((_NPAD, D), jnp.float32),
            pltpu.SemaphoreType.DMA,
            pltpu.SemaphoreType.DMA,
            pltpu.SemaphoreType.DMA,
            pltpu.SemaphoreType.DMA,
            pltpu.SemaphoreType.DMA,
            pltpu.SemaphoreType.DMA,
            pltpu.SemaphoreType.DMA,
            pltpu.SemaphoreType.DMA,
            pltpu.SemaphoreType.DMA,
            pltpu.SemaphoreType.DMA,
        ],
        compiler_params=pltpu.CompilerParams(needs_layout_passes=False),
    )(_edge_agg_kernel)
    return fn(xw, radial, src, dst, zeros_nd)


_BE = 1920

_BASIS_VALUES = np.linspace(0.0, MAX_RADIUS, NB + 2)[1:-1].astype(np.float32)
_BASIS_STEP = float(_BASIS_VALUES[1] - _BASIS_VALUES[0])
_BASIS_SCALE = float(1.14136 * np.exp(2.0) * np.sqrt(NB))


def _radial_body(len2_ref, w1_ref, b1_ref, w2_ref, out_ref):
    ln = jnp.sqrt(len2_ref[...] + 1e-12)
    values = (lax.broadcasted_iota(jnp.int32, (1, NB), 1).astype(jnp.float32)
              + 1.0) * _BASIS_STEP
    diff = (ln - values) / _BASIS_STEP
    d2 = jnp.clip(diff * diff, 0.0, 0.99999)
    y = jnp.where(jnp.abs(diff) < 1.0,
                  _BASIS_SCALE * jnp.exp(-1.0 / (1.0 - d2)), 0.0)
    h = y @ w1_ref[...] + b1_ref[...]
    h = h * jax.nn.sigmoid(h)
    radial = jnp.dot(h, w2_ref[...], preferred_element_type=jnp.float32)
    rows = (lax.broadcasted_iota(jnp.int32, (_BE, 1), 0)
            + pl.program_id(0) * _BE)
    out_ref[...] = jnp.where(rows < E, radial, 0.0)


def _radial(len2_pad, w1, b1, w2):
    len2c = len2_pad.reshape(_E_PAD, 1)
    b1r = b1.reshape(1, RN)
    return pl.pallas_call(
        _radial_body,
        grid=(_E_PAD // _BE,),
        in_specs=[
            pl.BlockSpec((_BE, 1), lambda i: (i, 0)),
            pl.BlockSpec((NB, RN), lambda i: (0, 0)),
            pl.BlockSpec((1, RN), lambda i: (0, 0)),
            pl.BlockSpec((RN, D), lambda i: (0, 0)),
        ],
        out_specs=pl.BlockSpec((_BE, D), lambda i: (i, 0)),
        out_shape=jax.ShapeDtypeStruct((_E_PAD, D), jnp.float32),
    )(len2c, w1, b1r, w2)


def _node_linear2_body(x1_ref, w1_ref, x2_ref, w2_ref, o1_ref, o2_ref):
    o1_ref[...] = jnp.dot(x1_ref[...], w1_ref[...],
                          preferred_element_type=jnp.float32)
    o2_ref[...] = jnp.dot(x2_ref[...], w2_ref[...],
                          preferred_element_type=jnp.float32)


def _node_linear2(x1, w1, x2, w2):
    oshape = jax.ShapeDtypeStruct((N, D), jnp.float32)
    return pl.pallas_call(
        _node_linear2_body,
        out_shape=(oshape, oshape),
    )(x1, w1, x2, w2)


def _net3_prep_body(o1_ref, o2_ref, p_ref, w_ref, xi_ref, xw_ref):
    p0 = p_ref[0, 0]
    xi = p0 * o1_ref[...] + (1.0 - p0) * o2_ref[...]
    xi_ref[...] = xi
    xw_ref[...] = jnp.dot(xi, w_ref[...], preferred_element_type=jnp.float32)


def _net3_prep(o1, o2, p0, w):
    return pl.pallas_call(
        _net3_prep_body,
        out_shape=(jax.ShapeDtypeStruct((N, D), jnp.float32),
                   jax.ShapeDtypeStruct((N, D), jnp.float32)),
    )(o1, o2, p0, w)


def _node_out_body(aggp_ref, x_ref, wout_ref, wself_ref, mask_ref, batch_ref,
                   out_ref):
    agg = (aggp_ref[pl.ds(0, N), :] + aggp_ref[pl.ds(_NPAD, N), :]) * INV_SQRT_NN
    t = jnp.tanh(jnp.dot(agg, wout_ref[...], preferred_element_type=jnp.float32)
                 + jnp.dot(x_ref[...], wself_ref[...],
                           preferred_element_type=jnp.float32))
    v = jnp.abs(t) * mask_ref[...]
    ss = jnp.sum(v * v, axis=1, keepdims=True)
    gids = lax.broadcasted_iota(jnp.int32, (N, NG), 1)
    oh = (batch_ref[...] == gids).astype(jnp.float32)
    gss = lax.dot_general(oh, ss, (((0,), (0,)), ((), ())),
                          preferred_element_type=jnp.float32)
    f = jnp.sqrt(gss + 1e-12)
    fac = jnp.dot(oh, f, preferred_element_type=jnp.float32)
    out_ref[...] = v / fac


def _node_out(agg_parts, x, w_out, w_self, mask, batch_col):
    return pl.pallas_call(
        _node_out_body,
        out_shape=jax.ShapeDtypeStruct((N, D), jnp.float32),
    )(agg_parts, x, w_out, w_self, mask, batch_col)


def kernel(pos, x, pos_final_state, x_final_state,
           pos_interpolated_transition_state, basis_mask, p, species, batch,
           edge_index, net1_W1, net1_b1, net1_W2, net1_W_lin, net1_W_self,
           net1_W_out, net2_W1, net2_b1, net2_W2, net2_W_lin, net2_W_self,
           net2_W_out, net3_W1, net3_b1, net3_W2, net3_W_lin, net3_W_self,
           net3_W_out):
    src = edge_index[0].astype(jnp.int32)
    dst = edge_index[1].astype(jnp.int32)
    batch_col = batch.astype(jnp.int32).reshape(N, 1)
    pos9 = jnp.concatenate(
        [pos.T, pos_final_state.T, pos_interpolated_transition_state.T],
        axis=0).reshape(9 * N)
    zeros_nd = jnp.zeros((_NPAD, D), jnp.float32)
    pad_i = jnp.zeros((_E_PAD - E,), jnp.int32)
    pad_f = jnp.zeros((_E_PAD - E,), jnp.float32)
    src_p = jnp.concatenate([src, pad_i])
    dst_p = jnp.concatenate([dst, pad_i])
    p0 = p[0:1].reshape(1, 1)

    len2 = _edge_len2(pos9, src, dst)

    rad1 = _radial(jnp.concatenate([len2[0:E], pad_f]),
                   net1_W1, net1_b1, net1_W2)
    rad2 = _radial(jnp.concatenate([len2[E:2 * E], pad_f]),
                   net2_W1, net2_b1, net2_W2)
    rad3 = _radial(jnp.concatenate([len2[2 * E:3 * E], pad_f]),
                   net3_W1, net3_b1, net3_W2)

    xw1, xw2 = _node_linear2(x, net1_W_lin, x_final_state, net2_W_lin)
    agg1 = _edge_agg(xw1, rad1, src_p, dst_p, zeros_nd)
    o1 = _node_out(agg1, x, net1_W_out, net1_W_self, basis_mask, batch_col)

    agg2 = _edge_agg(xw2, rad2, src_p, dst_p, zeros_nd)
    o2 = _node_out(agg2, x_final_state, net2_W_out, net2_W_self, basis_mask,
                   batch_col)

    xi, xw3 = _net3_prep(o1, o2, p0, net3_W_lin)
    agg3 = _edge_agg(xw3, rad3, src_p, dst_p, zeros_nd)
    o3 = _node_out(agg3, xi, net3_W_out, net3_W_self, basis_mask, batch_col)
    return o3


# speedup vs baseline: 1.0063x; 1.0063x over previous
import functools

import jax
import jax.numpy as jnp
import numpy as np
from jax import lax
from jax.experimental import pallas as pl
from jax.experimental.pallas import tpu as pltpu
from jax.experimental.pallas import tpu_sc as plsc

N = 10000
E = 320000
D = 128
NB = 10
RN = 64
NG = 16
MAX_RADIUS = 5.0
INV_SQRT_NN = 1.0 / np.sqrt(32.0)

_NC = 2
_NS = 16
_NW = _NC * _NS


_LEN_CH = 2000
_EPW = E // _NW


def _edge_len2_kernel(pos9_hbm, src_hbm, dst_hbm, out_hbm, pos_v, src_v,
                      dst_v, ob0_v, ob1_v, ob2_v):
    c = lax.axis_index("c")
    s = lax.axis_index("s")
    pltpu.sync_copy(pos9_hbm, pos_v)
    nch = _EPW // _LEN_CH
    base0 = (s * _NC + c) * _EPW
    obs = (ob0_v, ob1_v, ob2_v)

    def chunk_body(ci, carry):
        base = base0 + ci * _LEN_CH
        pltpu.sync_copy(src_hbm.at[pl.ds(base, _LEN_CH)], src_v)
        pltpu.sync_copy(dst_hbm.at[pl.ds(base, _LEN_CH)], dst_v)

        def vec_body(vi, c2):
            off = vi * 16
            si = src_v[pl.ds(off, 16)]
            di = dst_v[pl.ds(off, 16)]
            for t in range(3):
                acc = jnp.zeros((16,), jnp.float32)
                for comp in range(3):
                    bias = jnp.full((16,), (t * 3 + comp) * N, jnp.int32)
                    a = plsc.load_gather(pos_v, [bias + di])
                    b = plsc.load_gather(pos_v, [bias + si])
                    dv = a - b
                    acc = acc + dv * dv
                obs[t][pl.ds(off, 16)] = acc
            return c2

        lax.fori_loop(0, _LEN_CH // 16, vec_body, 0)
        for t in range(3):
            pltpu.sync_copy(obs[t], out_hbm.at[pl.ds(t * E + base, _LEN_CH)])
        return carry

    lax.fori_loop(0, nch, chunk_body, 0)


def _edge_len2(pos9, src, dst):
    mesh = plsc.VectorSubcoreMesh(core_axis_name="c", subcore_axis_name="s")
    fn = functools.partial(
        pl.kernel,
        mesh=mesh,
        out_type=jax.ShapeDtypeStruct((3 * E,), jnp.float32),
        scratch_types=[
            pltpu.VMEM((9 * N,), jnp.float32),
            pltpu.VMEM((_LEN_CH,), jnp.int32),
            pltpu.VMEM((_LEN_CH,), jnp.int32),
            pltpu.VMEM((_LEN_CH,), jnp.float32),
            pltpu.VMEM((_LEN_CH,), jnp.float32),
            pltpu.VMEM((_LEN_CH,), jnp.float32),
        ],
        compiler_params=pltpu.CompilerParams(needs_layout_passes=False),
    )(_edge_len2_kernel)
    return fn(pos9, src, dst)


_BLK = 96
_E_PAD = 322560
_MB0 = 140
_MB1 = 70
_NPAD = 10112
_RPT = _NPAD // _NS


def _edge_agg_kernel(xw_hbm, rad_hbm, src_hbm, dst_hbm, zeros_hbm, out_hbm,
                     si0, si1, di0, di1, xr0, xr1, rd0, rd1, agg_sh,
                     sg0, sg1, ss0, ss1, sd0, sd1, sr0, sr1, sc0, sc1):
    c = lax.axis_index("c")
    s = lax.axis_index("s")
    wid = s * _NC + c
    pltpu.sync_copy(zeros_hbm.at[pl.ds(s * _RPT, _RPT)],
                    agg_sh.at[pl.ds(s * _RPT, _RPT)])
    plsc.subcore_barrier()

    si = (si0, si1)
    di = (di0, di1)
    xr = (xr0, xr1)
    rd = (rd0, rd1)
    sg = (sg0, sg1)
    ss = (ss0, ss1)
    sd = (sd0, sd1)
    sr = (sr0, sr1)
    sc = (sc0, sc1)
    mb = jnp.where(c == 0, _MB0, _MB1)
    base0 = jnp.where(c == 0, s * _MB0, _NS * _MB0 + s * _MB1) * _BLK

    def _multiply(xr_v, rd_v):
        def row_body(i, c2):
            for j in range(D // 16):
                sl = pl.ds(j * 16, 16)
                xr_v[i, sl] = xr_v[i, sl] * rd_v[i, sl]
            return c2

        lax.fori_loop(0, _BLK, row_body, 0)

    pltpu.sync_copy(src_hbm.at[pl.ds(base0, _BLK)], si0)
    pltpu.async_copy(xw_hbm.at[si0], xr0, sg0)
    pltpu.async_copy(rad_hbm.at[pl.ds(base0, _BLK)], rd0, sr0)
    pltpu.async_copy(dst_hbm.at[pl.ds(base0, _BLK)], di0, sd0)
    pltpu.async_copy(src_hbm.at[pl.ds(base0 + _BLK, _BLK)], si1, ss1)
    pltpu.async_copy(rad_hbm.at[pl.ds(base0 + _BLK, _BLK)], rd1, sr1)

    def step(kk, carry):
        for slot in (0, 1):
            nxt = slot ^ 1
            k = kk * 2 + slot
            base = base0 + k * _BLK
            pltpu.make_async_copy(xw_hbm.at[si[slot]], xr[slot],
                                  sg[slot]).wait()

            @pl.when(k + 1 < mb)
            def _():
                pltpu.make_async_copy(src_hbm.at[pl.ds(base + _BLK, _BLK)],
                                      si[nxt], ss[nxt]).wait()

                @pl.when(k >= 1)
                def _():
                    pltpu.make_async_copy(xr[nxt], agg_sh.at[di[nxt]],
                                          sc[nxt]).wait()

                pltpu.async_copy(xw_hbm.at[si[nxt]], xr[nxt], sg[nxt])
                pltpu.async_copy(dst_hbm.at[pl.ds(base + _BLK, _BLK)],
                                 di[nxt], sd[nxt])

            @pl.when(k + 2 < mb)
            def _():
                pltpu.async_copy(src_hbm.at[pl.ds(base + 2 * _BLK, _BLK)],
                                 si[slot], ss[slot])

            pltpu.make_async_copy(rad_hbm.at[pl.ds(base, _BLK)], rd[slot],
                                  sr[slot]).wait()
            _multiply(xr[slot], rd[slot])

            @pl.when(k + 2 < mb)
            def _():
                pltpu.async_copy(rad_hbm.at[pl.ds(base + 2 * _BLK, _BLK)],
                                 rd[slot], sr[slot])

            pltpu.make_async_copy(dst_hbm.at[pl.ds(base, _BLK)], di[slot],
                                  sd[slot]).wait()
            pltpu.async_copy(xr[slot], agg_sh.at[di[slot]], sc[slot],
                             add=True)

        return carry

    lax.fori_loop(0, mb // 2, step, 0)

    pltpu.make_async_copy(xr0, agg_sh.at[di0], sc0).wait()
    pltpu.make_async_copy(xr1, agg_sh.at[di1], sc1).wait()

    plsc.subcore_barrier()
    pltpu.sync_copy(agg_sh.at[pl.ds(s * _RPT, _RPT)],
                    out_hbm.at[pl.ds(c * _NPAD + s * _RPT, _RPT)])


def _edge_agg(xw, radial, src, dst, zeros_nd):
    mesh = plsc.VectorSubcoreMesh(core_axis_name="c", subcore_axis_name="s")
    fn = functools.partial(
        pl.kernel,
        mesh=mesh,
        out_type=jax.ShapeDtypeStruct((2 * _NPAD, D), jnp.float32),
        scratch_types=[
            pltpu.VMEM((_BLK,), jnp.int32),
            pltpu.VMEM((_BLK,), jnp.int32),
            pltpu.VMEM((_BLK,), jnp.int32),
            pltpu.VMEM((_BLK,), jnp.int32),
            pltpu.VMEM((_BLK, D), jnp.float32),
            pltpu.VMEM((_BLK, D), jnp.float32),
            pltpu.VMEM((_BLK, D), jnp.float32),
            pltpu.VMEM((_BLK, D), jnp.float32),
            pltpu.VMEM_SHARED((_NPAD, D), jnp.float32),
            pltpu.SemaphoreType.DMA,
            pltpu.SemaphoreType.DMA,
            pltpu.SemaphoreType.DMA,
            pltpu.SemaphoreType.DMA,
            pltpu.SemaphoreType.DMA,
            pltpu.SemaphoreType.DMA,
            pltpu.SemaphoreType.DMA,
            pltpu.SemaphoreType.DMA,
            pltpu.SemaphoreType.DMA,
            pltpu.SemaphoreType.DMA,
        ],
        compiler_params=pltpu.CompilerParams(needs_layout_passes=False),
    )(_edge_agg_kernel)
    return fn(xw, radial, src, dst, zeros_nd)


_BE = 1920

_BASIS_VALUES = np.linspace(0.0, MAX_RADIUS, NB + 2)[1:-1].astype(np.float32)
_BASIS_STEP = float(_BASIS_VALUES[1] - _BASIS_VALUES[0])
_BASIS_SCALE = float(1.14136 * np.exp(2.0) * np.sqrt(NB))


def _radial_body(len2_ref, w1_ref, b1_ref, w2_ref, out_ref):
    ln = jnp.sqrt(len2_ref[...] + 1e-12)
    values = (lax.broadcasted_iota(jnp.int32, (1, NB), 1).astype(jnp.float32)
              + 1.0) * _BASIS_STEP
    diff = (ln - values) / _BASIS_STEP
    d2 = jnp.clip(diff * diff, 0.0, 0.99999)
    y = jnp.where(jnp.abs(diff) < 1.0,
                  _BASIS_SCALE * jnp.exp(-1.0 / (1.0 - d2)), 0.0)
    h = y @ w1_ref[...] + b1_ref[...]
    h = h * jax.nn.sigmoid(h)
    radial = jnp.dot(h, w2_ref[...], preferred_element_type=jnp.float32)
    rows = (lax.broadcasted_iota(jnp.int32, (_BE, 1), 0)
            + pl.program_id(0) * _BE)
    out_ref[...] = jnp.where(rows < E, radial, 0.0)


def _radial(len2_pad, w1, b1, w2):
    len2c = len2_pad.reshape(_E_PAD, 1)
    b1r = b1.reshape(1, RN)
    return pl.pallas_call(
        _radial_body,
        grid=(_E_PAD // _BE,),
        in_specs=[
            pl.BlockSpec((_BE, 1), lambda i: (i, 0)),
            pl.BlockSpec((NB, RN), lambda i: (0, 0)),
            pl.BlockSpec((1, RN), lambda i: (0, 0)),
            pl.BlockSpec((RN, D), lambda i: (0, 0)),
        ],
        out_specs=pl.BlockSpec((_BE, D), lambda i: (i, 0)),
        out_shape=jax.ShapeDtypeStruct((_E_PAD, D), jnp.float32),
    )(len2c, w1, b1r, w2)


def _node_linear2_body(x1_ref, w1_ref, x2_ref, w2_ref, o1_ref, o2_ref):
    o1_ref[...] = jnp.dot(x1_ref[...], w1_ref[...],
                          preferred_element_type=jnp.float32)
    o2_ref[...] = jnp.dot(x2_ref[...], w2_ref[...],
                          preferred_element_type=jnp.float32)


def _node_linear2(x1, w1, x2, w2):
    oshape = jax.ShapeDtypeStruct((N, D), jnp.float32)
    return pl.pallas_call(
        _node_linear2_body,
        out_shape=(oshape, oshape),
    )(x1, w1, x2, w2)


def _net3_prep_body(o1_ref, o2_ref, p_ref, w_ref, xi_ref, xw_ref):
    p0 = p_ref[0, 0]
    xi = p0 * o1_ref[...] + (1.0 - p0) * o2_ref[...]
    xi_ref[...] = xi
    xw_ref[...] = jnp.dot(xi, w_ref[...], preferred_element_type=jnp.float32)


def _net3_prep(o1, o2, p0, w):
    return pl.pallas_call(
        _net3_prep_body,
        out_shape=(jax.ShapeDtypeStruct((N, D), jnp.float32),
                   jax.ShapeDtypeStruct((N, D), jnp.float32)),
    )(o1, o2, p0, w)


def _node_out_body(aggp_ref, x_ref, wout_ref, wself_ref, mask_ref, batch_ref,
                   out_ref):
    agg = (aggp_ref[pl.ds(0, N), :] + aggp_ref[pl.ds(_NPAD, N), :]) * INV_SQRT_NN
    t = jnp.tanh(jnp.dot(agg, wout_ref[...], preferred_element_type=jnp.float32)
                 + jnp.dot(x_ref[...], wself_ref[...],
                           preferred_element_type=jnp.float32))
    v = jnp.abs(t) * mask_ref[...]
    ss = jnp.sum(v * v, axis=1, keepdims=True)
    gids = lax.broadcasted_iota(jnp.int32, (N, NG), 1)
    oh = (batch_ref[...] == gids).astype(jnp.float32)
    gss = lax.dot_general(oh, ss, (((0,), (0,)), ((), ())),
                          preferred_element_type=jnp.float32)
    f = jnp.sqrt(gss + 1e-12)
    fac = jnp.dot(oh, f, preferred_element_type=jnp.float32)
    out_ref[...] = v / fac


def _node_out(agg_parts, x, w_out, w_self, mask, batch_col):
    return pl.pallas_call(
        _node_out_body,
        out_shape=jax.ShapeDtypeStruct((N, D), jnp.float32),
    )(agg_parts, x, w_out, w_self, mask, batch_col)


def kernel(pos, x, pos_final_state, x_final_state,
           pos_interpolated_transition_state, basis_mask, p, species, batch,
           edge_index, net1_W1, net1_b1, net1_W2, net1_W_lin, net1_W_self,
           net1_W_out, net2_W1, net2_b1, net2_W2, net2_W_lin, net2_W_self,
           net2_W_out, net3_W1, net3_b1, net3_W2, net3_W_lin, net3_W_self,
           net3_W_out):
    src = edge_index[0].astype(jnp.int32)
    dst = edge_index[1].astype(jnp.int32)
    batch_col = batch.astype(jnp.int32).reshape(N, 1)
    pos9 = jnp.concatenate(
        [pos.T, pos_final_state.T, pos_interpolated_transition_state.T],
        axis=0).reshape(9 * N)
    zeros_nd = jnp.zeros((_NPAD, D), jnp.float32)
    pad_i = jnp.zeros((_E_PAD - E,), jnp.int32)
    pad_f = jnp.zeros((_E_PAD - E,), jnp.float32)
    src_p = jnp.concatenate([src, pad_i])
    dst_p = jnp.concatenate([dst, pad_i])
    p0 = p[0:1].reshape(1, 1)

    len2 = _edge_len2(pos9, src, dst)

    rad1 = _radial(jnp.concatenate([len2[0:E], pad_f]),
                   net1_W1, net1_b1, net1_W2)
    rad2 = _radial(jnp.concatenate([len2[E:2 * E], pad_f]),
                   net2_W1, net2_b1, net2_W2)
    rad3 = _radial(jnp.concatenate([len2[2 * E:3 * E], pad_f]),
                   net3_W1, net3_b1, net3_W2)

    xw1, xw2 = _node_linear2(x, net1_W_lin, x_final_state, net2_W_lin)
    agg1 = _edge_agg(xw1, rad1, src_p, dst_p, zeros_nd)
    o1 = _node_out(agg1, x, net1_W_out, net1_W_self, basis_mask, batch_col)

    agg2 = _edge_agg(xw2, rad2, src_p, dst_p, zeros_nd)
    o2 = _node_out(agg2, x_final_state, net2_W_out, net2_W_self, basis_mask,
                   batch_col)

    xi, xw3 = _net3_prep(o1, o2, p0, net3_W_lin)
    agg3 = _edge_agg(xw3, rad3, src_p, dst_p, zeros_nd)
    o3 = _node_out(agg3, xi, net3_W_out, net3_W_self, basis_mask, batch_col)
    return o3
